# Initial kernel scaffold; baseline (speedup 1.0000x reference)
import jax
import jax.numpy as jnp
from jax.experimental import pallas as pl


def kernel(x, edge_index_seq, edge_index_shape, edge_index_color, batch, shape_emb, color_emb, W0, b0, rel1_Wl, rel1_bl, rel1_Wr, rel2_Wl, rel2_bl, rel2_Wr, Wout, bout):
    raise NotImplementedError("write your pallas kernel here")


# speedup vs baseline: 1.0104x; 1.0104x over previous
import functools

import jax
import jax.numpy as jnp
from jax.experimental import pallas as pl
from jax.experimental.pallas import tpu as pltpu

N = 50000
E = 800000
EMB = 32
HID = 64
NCLS = 2
NG = 1024

BN = 2000


def _mlp0_body(sh, co, w, b, out):
    x = jnp.concatenate([sh[...], co[...]], axis=1)
    out[...] = jax.nn.relu(
        jnp.dot(x, w[...], preferred_element_type=jnp.float32) + b[...])


def _mlp0(sh, co, wT, b):
    grid = N // BN
    return pl.pallas_call(
        _mlp0_body,
        grid=(grid,),
        in_specs=[
            pl.BlockSpec((BN, EMB), lambda i: (i, 0)),
            pl.BlockSpec((BN, EMB), lambda i: (i, 0)),
            pl.BlockSpec((2 * EMB, HID), lambda i: (0, 0)),
            pl.BlockSpec((1, HID), lambda i: (0, 0)),
        ],
        out_specs=pl.BlockSpec((BN, HID), lambda i: (i, 0)),
        out_shape=jax.ShapeDtypeStruct((N, HID), jnp.float32),
    )(sh, co, wT, b.reshape(1, HID))


def _layer_body(a0, a1, a2, c0, c1, c2, h, w, b, out):
    m0 = a0[...] * (1.0 / jnp.maximum(c0[...], 1.0))
    m1 = a1[...] * (1.0 / jnp.maximum(c1[...], 1.0))
    m2 = a2[...] * (1.0 / jnp.maximum(c2[...], 1.0))
    x = jnp.concatenate([m0, m1, m2, h[...]], axis=1)
    out[...] = jax.nn.relu(
        jnp.dot(x, w[...], preferred_element_type=jnp.float32) + b[...])


def _layer(aggs, cnts, h, wT_cat, b_sum):
    grid = N // BN
    vec = pl.BlockSpec((BN, HID), lambda i: (i, 0))
    col = pl.BlockSpec((BN, 1), lambda i: (i, 0))
    return pl.pallas_call(
        _layer_body,
        grid=(grid,),
        in_specs=[vec, vec, vec, col, col, col, vec,
                  pl.BlockSpec((4 * HID, HID), lambda i: (0, 0)),
                  pl.BlockSpec((1, HID), lambda i: (0, 0))],
        out_specs=vec,
        out_shape=jax.ShapeDtypeStruct((N, HID), jnp.float32),
    )(*aggs, *cnts, h, wT_cat, b_sum.reshape(1, HID))


def _readout_body(g, c, w, b, out):
    pooled = g[...] * (1.0 / jnp.maximum(c[...], 1.0))
    out[...] = jnp.dot(pooled, w[...], preferred_element_type=jnp.float32) + b[...]


def _readout(gacc, gcnt, woutT, bout):
    return pl.pallas_call(
        _readout_body,
        in_specs=[pl.BlockSpec((NG, HID), lambda: (0, 0)),
                  pl.BlockSpec((NG, 1), lambda: (0, 0)),
                  pl.BlockSpec((HID, NCLS), lambda: (0, 0)),
                  pl.BlockSpec((1, NCLS), lambda: (0, 0))],
        out_specs=pl.BlockSpec((NG, NCLS), lambda: (0, 0)),
        out_shape=jax.ShapeDtypeStruct((NG, NCLS), jnp.float32),
    )(gacc, gcnt, woutT, bout.reshape(1, NCLS))


def _agg(h, src, dst):
    return jax.ops.segment_sum(h[src], dst, num_segments=N)


def kernel(x, edge_index_seq, edge_index_shape, edge_index_color, batch,
           shape_emb, color_emb, W0, b0,
           rel1_Wl, rel1_bl, rel1_Wr,
           rel2_Wl, rel2_bl, rel2_Wr,
           Wout, bout):
    eis = [edge_index_seq, edge_index_shape, edge_index_color]
    ones_e = jnp.ones((E,), jnp.float32)
    cnts = [jax.ops.segment_sum(ones_e, ei[1], num_segments=N).reshape(N, 1)
            for ei in eis]

    sh = shape_emb[x[:, 0]]
    co = color_emb[x[:, 1]]
    h = _mlp0(sh, co, W0.T, b0)

    for Wl, bl, Wr in ((rel1_Wl, rel1_bl, rel1_Wr),
                       (rel2_Wl, rel2_bl, rel2_Wr)):
        aggs = [_agg(h, ei[0], ei[1]) for ei in eis]
        wT_cat = jnp.concatenate(
            [Wl[0].T, Wl[1].T, Wl[2].T, (Wr[0] + Wr[1] + Wr[2]).T], axis=0)
        h = _layer(aggs, cnts, h, wT_cat, bl[0] + bl[1] + bl[2])

    gacc = jax.ops.segment_sum(h, batch, num_segments=NG)
    gcnt = jax.ops.segment_sum(jnp.ones((N,), jnp.float32), batch,
                               num_segments=NG).reshape(NG, 1)
    return _readout(gacc, gcnt, Wout.T, bout)


# speedup vs baseline: 4.6815x; 4.6333x over previous
import jax
import jax.numpy as jnp
from jax import lax
from jax.experimental import pallas as pl
from jax.experimental.pallas import tpu as pltpu
from jax.experimental.pallas import tpu_sc as plsc

N = 50000
E = 800000
EMB = 32
HID = 64
NCLS = 2
NG = 1024

F32 = jnp.float32
QW = 16

GL = 128
GPC = 8
CHUNK = GL * GPC
CPT_E = 49
E_PAD = 16 * CPT_E * CHUNK
EROWS = E_PAD // GL

NPOOL = 65536
CPT_P = NPOOL // (16 * CHUNK)
PROWS = NPOOL // GL

NPT = N // 16
ZR = 625
N_ACC = N + 8
NG_ACC = 1040

BN = 2000

_MESH = plsc.VectorSubcoreMesh(core_axis_name="c", subcore_axis_name="s")
_SC_PARAMS = pltpu.CompilerParams(use_tc_tiling_on_sc=False)


def _fill(buf, nrows, ncols, vec):
    def body(i, _):
        for j in range(ncols // 16):
            buf[i, pl.ds(j * 16, 16)] = vec
        return 0

    lax.fori_loop(0, nrows, body, 0)


def _emb_body(sh_tab, co_tab, xsh, xco, out_sh, out_co, idx, rows, sem):
    c = lax.axis_index("c")
    s = lax.axis_index("s")

    def run(tab, src2d, out):
        def chunk(ch, _):
            row0 = (s * CPT_P + ch) * GPC
            pltpu.sync_copy(src2d.at[pl.ds(row0, GPC), :], idx)
            hs = [pltpu.async_copy(tab.at[idx.at[j]],
                                   rows.at[pl.ds(j * GL, GL), :], sem)
                  for j in range(GPC)]
            for h in hs:
                h.wait()
            pltpu.sync_copy(rows, out.at[pl.ds(row0 * GL, CHUNK), :])
            return 0

        lax.fori_loop(0, CPT_P, chunk, 0)

    @pl.when(c == 0)
    def _():
        run(sh_tab, xsh, out_sh)

    @pl.when(c == 1)
    def _():
        run(co_tab, xco, out_co)


def _emb(sh_tab, co_tab, xsh2d, xco2d):
    return pl.kernel(
        _emb_body,
        out_type=[jax.ShapeDtypeStruct((NPOOL, EMB), F32),
                  jax.ShapeDtypeStruct((NPOOL, EMB), F32)],
        mesh=_MESH,
        compiler_params=_SC_PARAMS,
        scratch_types=[
            pltpu.VMEM((GPC, GL), jnp.int32),
            pltpu.VMEM((CHUNK, EMB), F32),
            pltpu.SemaphoreType.DMA,
        ],
    )(sh_tab, co_tab, xsh2d, xco2d)


def _agg_body(src0, dst0, src1, dst1, src2, dst2, hq0, hq1, hq2, hq3,
              o0_lo, o1_lo, o2_lo, o0_hi, o1_hi, o2_hi,
              sidx, didx, rows, zbuf, acc, sem, sem2):
    c = lax.axis_index("c")
    s = lax.axis_index("s")
    _fill(zbuf, ZR, QW, jnp.zeros((16,), F32))
    srcs = (src0, src1, src2)
    dsts = (dst0, dst1, dst2)

    def run(hA, hB, outs):
        for r in range(3):
            for hq, col in ((hA, 0), (hB, QW)):
                for k in range(5):
                    pltpu.sync_copy(
                        zbuf, acc.at[pl.ds(s * NPT + k * ZR, ZR), :])
                plsc.subcore_barrier()

                def chunk(ch, _):
                    row0 = (s * CPT_E + ch) * GPC
                    pltpu.sync_copy(srcs[r].at[pl.ds(row0, GPC), :], sidx)
                    pltpu.sync_copy(dsts[r].at[pl.ds(row0, GPC), :], didx)
                    hs = [pltpu.async_copy(hq.at[sidx.at[j]],
                                           rows.at[pl.ds(j * GL, GL), :],
                                           sem)
                          for j in range(GPC)]
                    for h in hs:
                        h.wait()
                    hs2 = [pltpu.async_copy(rows.at[pl.ds(j * GL, GL), :],
                                            acc.at[didx.at[j]], sem2,
                                            add=True)
                           for j in range(GPC)]
                    for h in hs2:
                        h.wait()
                    return 0

                lax.fori_loop(0, CPT_E, chunk, 0)
                plsc.subcore_barrier()
                pltpu.sync_copy(acc.at[pl.ds(s * NPT, NPT), :],
                                outs[r].at[pl.ds(s * NPT, NPT),
                                           pl.ds(col, QW)])
                plsc.subcore_barrier()

    @pl.when(c == 0)
    def _():
        run(hq0, hq1, (o0_lo, o1_lo, o2_lo))

    @pl.when(c == 1)
    def _():
        run(hq2, hq3, (o0_hi, o1_hi, o2_hi))


def _agg(edges2d, hqs):
    out = jax.ShapeDtypeStruct((N, EMB), F32)
    return pl.kernel(
        _agg_body,
        out_type=[out] * 6,
        mesh=_MESH,
        compiler_params=_SC_PARAMS,
        scratch_types=[
            pltpu.VMEM((GPC, GL), jnp.int32),
            pltpu.VMEM((GPC, GL), jnp.int32),
            pltpu.VMEM((CHUNK, QW), F32),
            pltpu.VMEM((ZR, QW), F32),
            pltpu.VMEM_SHARED((N_ACC, QW), F32),
            pltpu.SemaphoreType.DMA,
            pltpu.SemaphoreType.DMA,
        ],
    )(edges2d[0][0], edges2d[0][1], edges2d[1][0], edges2d[1][1],
      edges2d[2][0], edges2d[2][1], *hqs)


NH = 50176
HPT = NH // 16


def _cnt_body(dst0, dst1, dst2, dstp, o0, o1, o2, og,
              didx, ones1, zb1, hist, sem2):
    c = lax.axis_index("c")
    s = lax.axis_index("s")
    wid = c * 16 + s
    one = jnp.ones((16,), F32)
    zero = jnp.zeros((16,), F32)

    def fill1(i, _):
        ones1[pl.ds(i * 16, 16)] = one
        return 0

    lax.fori_loop(0, GL // 16, fill1, 0)

    def fillz(i, _):
        zb1[pl.ds(i * 16, 16)] = zero
        return 0

    lax.fori_loop(0, HPT // 16, fillz, 0)

    dsts = (dst0, dst1, dst2)
    outs = (o0, o1, o2)

    def add_chunk(dst2d, ch_abs):
        row0 = ch_abs * GPC
        pltpu.sync_copy(dst2d.at[pl.ds(row0, GPC), :], didx)
        hs = [pltpu.async_copy(ones1, hist.at[didx.at[j]], sem2, add=True)
              for j in range(GPC)]
        for h in hs:
            h.wait()

    n_ch = jnp.where(wid < 16, 25, 24)
    base_ch = wid * 24 + jnp.minimum(wid, 16)
    for r in range(3):
        pltpu.sync_copy(zb1, hist.at[pl.ds(s * HPT, HPT)])
        plsc.subcore_barrier()

        def chunk(i, _):
            add_chunk(dsts[r], base_ch + i)
            return 0

        lax.fori_loop(0, n_ch, chunk, 0)
        plsc.subcore_barrier()
        pltpu.sync_copy(hist.at[pl.ds(s * HPT, HPT)],
                        outs[r].at[c, pl.ds(s * HPT, HPT)])
        plsc.subcore_barrier()

    pltpu.sync_copy(zb1.at[pl.ds(0, 128)], hist.at[pl.ds(s * 128, 128)])
    plsc.subcore_barrier()

    def chunkb(i, _):
        add_chunk(dstp, wid * 2 + i)
        return 0

    lax.fori_loop(0, 2, chunkb, 0)
    plsc.subcore_barrier()
    pltpu.sync_copy(hist.at[pl.ds(s * 64, 64)], og.at[c, pl.ds(s * 64, 64)])


def _cnt(dst2d_rels, dstp2d):
    return pl.kernel(
        _cnt_body,
        out_type=[jax.ShapeDtypeStruct((2, NH), F32)] * 3
        + [jax.ShapeDtypeStruct((2, NG), F32)],
        mesh=_MESH,
        compiler_params=_SC_PARAMS,
        scratch_types=[
            pltpu.VMEM((GPC, GL), jnp.int32),
            pltpu.VMEM((GL,), F32),
            pltpu.VMEM((HPT,), F32),
            pltpu.VMEM_SHARED((NH,), F32),
            pltpu.SemaphoreType.DMA,
        ],
    )(dst2d_rels[0], dst2d_rels[1], dst2d_rels[2], dstp2d)


def _pool_body(srcp, dstp, hq0, hq1, hq2, hq3, o_lo, o_hi,
               sidx, didx, rows, zbuf, acc, sem, sem2):
    c = lax.axis_index("c")
    s = lax.axis_index("s")
    _fill(zbuf, 65, QW, jnp.zeros((16,), F32))

    def run(hA, hB, out):
        for hq, col in ((hA, 0), (hB, QW)):
            pltpu.sync_copy(zbuf, acc.at[pl.ds(s * 65, 65), :])
            plsc.subcore_barrier()

            def chunk(ch, _):
                row0 = (s * CPT_P + ch) * GPC
                pltpu.sync_copy(srcp.at[pl.ds(row0, GPC), :], sidx)
                pltpu.sync_copy(dstp.at[pl.ds(row0, GPC), :], didx)
                hs = [pltpu.async_copy(hq.at[sidx.at[j]],
                                       rows.at[pl.ds(j * GL, GL), :], sem)
                      for j in range(GPC)]
                for h in hs:
                    h.wait()
                hs2 = [pltpu.async_copy(rows.at[pl.ds(j * GL, GL), :],
                                        acc.at[didx.at[j]], sem2, add=True)
                       for j in range(GPC)]
                for h in hs2:
                    h.wait()
                return 0

            lax.fori_loop(0, CPT_P, chunk, 0)
            plsc.subcore_barrier()
            pltpu.sync_copy(acc.at[pl.ds(s * 64, 64), :],
                            out.at[pl.ds(s * 64, 64), pl.ds(col, QW)])
            plsc.subcore_barrier()

    @pl.when(c == 0)
    def _():
        run(hq0, hq1, o_lo)

    @pl.when(c == 1)
    def _():
        run(hq2, hq3, o_hi)


def _pool(srcp2d, dstp2d, hqs):
    out = jax.ShapeDtypeStruct((NG, EMB), F32)
    return pl.kernel(
        _pool_body,
        out_type=[out, out],
        mesh=_MESH,
        compiler_params=_SC_PARAMS,
        scratch_types=[
            pltpu.VMEM((GPC, GL), jnp.int32),
            pltpu.VMEM((GPC, GL), jnp.int32),
            pltpu.VMEM((CHUNK, QW), F32),
            pltpu.VMEM((65, QW), F32),
            pltpu.VMEM_SHARED((NG_ACC, QW), F32),
            pltpu.SemaphoreType.DMA,
            pltpu.SemaphoreType.DMA,
        ],
    )(srcp2d, dstp2d, *hqs)


def _split_q(y, outs):
    for q in range(4):
        outs[q][...] = y[:, q * QW:(q + 1) * QW]


def _mlp0_body(sh, co, w, b, oq0, oq1, oq2, oq3):
    x = jnp.concatenate([sh[...], co[...]], axis=1)
    y = jax.nn.relu(
        jnp.dot(x, w[...], preferred_element_type=F32) + b[...])
    _split_q(y, (oq0, oq1, oq2, oq3))


def _mlp0(sh, co, wT, b):
    grid = N // BN
    half = pl.BlockSpec((BN, EMB), lambda i: (i, 0))
    quart = pl.BlockSpec((BN, QW), lambda i: (i, 0))
    return pl.pallas_call(
        _mlp0_body,
        grid=(grid,),
        in_specs=[half, half,
                  pl.BlockSpec((2 * EMB, HID), lambda i: (0, 0)),
                  pl.BlockSpec((1, HID), lambda i: (0, 0))],
        out_specs=[quart] * 4,
        out_shape=[jax.ShapeDtypeStruct((N, QW), F32)] * 4,
    )(sh, co, wT, b.reshape(1, HID))


def _layer_body(a0l, a0h, a1l, a1h, a2l, a2h, c0, c1, c2,
                hq0, hq1, hq2, hq3, w, b, oq0, oq1, oq2, oq3):
    r0 = 1.0 / jnp.maximum(c0[...], 1.0)
    r1 = 1.0 / jnp.maximum(c1[...], 1.0)
    r2 = 1.0 / jnp.maximum(c2[...], 1.0)
    x = jnp.concatenate(
        [a0l[...] * r0, a0h[...] * r0,
         a1l[...] * r1, a1h[...] * r1,
         a2l[...] * r2, a2h[...] * r2,
         hq0[...], hq1[...], hq2[...], hq3[...]], axis=1)
    y = jax.nn.relu(
        jnp.dot(x, w[...], preferred_element_type=F32) + b[...])
    _split_q(y, (oq0, oq1, oq2, oq3))


def _layer(aggs, cnts, hqs, wT_cat, b_sum):
    grid = N // BN
    half = pl.BlockSpec((BN, EMB), lambda i: (i, 0))
    quart = pl.BlockSpec((BN, QW), lambda i: (i, 0))
    col = pl.BlockSpec((BN, 1), lambda i: (i, 0))
    return pl.pallas_call(
        _layer_body,
        grid=(grid,),
        in_specs=[half] * 6 + [col] * 3 + [quart] * 4 + [
            pl.BlockSpec((4 * HID, HID), lambda i: (0, 0)),
            pl.BlockSpec((1, HID), lambda i: (0, 0))],
        out_specs=[quart] * 4,
        out_shape=[jax.ShapeDtypeStruct((N, QW), F32)] * 4,
    )(*aggs, *cnts, *hqs, wT_cat, b_sum.reshape(1, HID))


def _readout_body(gl_, gh, c, w, b, out):
    rc = 1.0 / jnp.maximum(c[...], 1.0)
    pooled = jnp.concatenate([gl_[...] * rc, gh[...] * rc], axis=1)
    out[...] = jnp.dot(pooled, w[...], preferred_element_type=F32) + b[...]


def _readout(gp_lo, gp_hi, gcnt, woutT, bout):
    return pl.pallas_call(
        _readout_body,
        in_specs=[pl.BlockSpec((NG, EMB), lambda: (0, 0)),
                  pl.BlockSpec((NG, EMB), lambda: (0, 0)),
                  pl.BlockSpec((NG, 1), lambda: (0, 0)),
                  pl.BlockSpec((HID, NCLS), lambda: (0, 0)),
                  pl.BlockSpec((1, NCLS), lambda: (0, 0))],
        out_specs=pl.BlockSpec((NG, NCLS), lambda: (0, 0)),
        out_shape=jax.ShapeDtypeStruct((NG, NCLS), F32),
    )(gp_lo, gp_hi, gcnt, woutT, bout.reshape(1, NCLS))


def kernel(x, edge_index_seq, edge_index_shape, edge_index_color, batch,
           shape_emb, color_emb, W0, b0,
           rel1_Wl, rel1_bl, rel1_Wr,
           rel2_Wl, rel2_bl, rel2_Wr,
           Wout, bout):
    eis = (edge_index_seq, edge_index_shape, edge_index_color)

    pad_e = E_PAD - E
    edges2d = []
    for ei in eis:
        src = jnp.concatenate([ei[0], jnp.zeros((pad_e,), ei.dtype)])
        dst = jnp.concatenate([ei[1], jnp.full((pad_e,), N, ei.dtype)])
        edges2d.append((src.reshape(EROWS, GL).astype(jnp.int32),
                        dst.reshape(EROWS, GL).astype(jnp.int32)))

    pad_n = NPOOL - N
    xi = x.astype(jnp.int32)
    xsh2d = jnp.concatenate([xi[:, 0], jnp.zeros((pad_n,), jnp.int32)]
                            ).reshape(PROWS, GL)
    xco2d = jnp.concatenate([xi[:, 1], jnp.zeros((pad_n,), jnp.int32)]
                            ).reshape(PROWS, GL)
    srcp2d = jnp.concatenate([jnp.arange(N, dtype=jnp.int32),
                              jnp.zeros((pad_n,), jnp.int32)]
                             ).reshape(PROWS, GL)
    dstp2d = jnp.concatenate([batch.astype(jnp.int32),
                              jnp.full((pad_n,), NG, jnp.int32)]
                             ).reshape(PROWS, GL)

    esh, eco = _emb(shape_emb, color_emb, xsh2d, xco2d)
    cnt_parts = _cnt([e[1] for e in edges2d], dstp2d)
    cnts = [(p[0, :N] + p[1, :N]).reshape(N, 1) for p in cnt_parts[:3]]
    gcnt = (cnt_parts[3][0] + cnt_parts[3][1]).reshape(NG, 1)

    hqs = _mlp0(esh[:N], eco[:N], W0.T, b0)

    for Wl, bl, Wr in ((rel1_Wl, rel1_bl, rel1_Wr),
                       (rel2_Wl, rel2_bl, rel2_Wr)):
        aggs = _agg(edges2d, hqs)
        aggs = (aggs[0], aggs[3], aggs[1], aggs[4], aggs[2], aggs[5])
        wT_cat = jnp.concatenate(
            [Wl[0].T, Wl[1].T, Wl[2].T, (Wr[0] + Wr[1] + Wr[2]).T], axis=0)
        hqs = _layer(aggs, cnts, hqs, wT_cat, bl[0] + bl[1] + bl[2])

    gp_lo, gp_hi = _pool(srcp2d, dstp2d, hqs)
    return _readout(gp_lo, gp_hi, gcnt, Wout.T, bout)


# speedup vs baseline: 6.4526x; 1.3783x over previous
import jax
import jax.numpy as jnp
from jax import lax
from jax.experimental import pallas as pl
from jax.experimental.pallas import tpu as pltpu
from jax.experimental.pallas import tpu_sc as plsc

N = 50000
E = 800000
EMB = 32
HID = 64
NCLS = 2
NG = 1024

F32 = jnp.float32
QW = 16

GL = 128
GPC = 8
CHUNK = GL * GPC
CPT_E = 50
E_PAD = 16 * CPT_E * CHUNK
EROWS = E_PAD // GL

NPOOL = 65536
CPT_P = NPOOL // (16 * CHUNK)
PROWS = NPOOL // GL

NPT = N // 16
ZR = 625
N_ACC = N + 16
NG_ACC = 1040

BN = 2000

_MESH = plsc.VectorSubcoreMesh(core_axis_name="c", subcore_axis_name="s")
_SC_PARAMS = pltpu.CompilerParams(use_tc_tiling_on_sc=False)


def _fill(buf, nrows, ncols, vec):
    def body(i, _):
        for j in range(ncols // 16):
            buf[i, pl.ds(j * 16, 16)] = vec
        return 0

    lax.fori_loop(0, nrows, body, 0)


def _emb_body(sh_tab, co_tab, xsh, xco, out_sh, out_co, idx, rows, sem):
    c = lax.axis_index("c")
    s = lax.axis_index("s")

    def run(tab, src2d, out):
        def chunk(ch, _):
            row0 = (s * CPT_P + ch) * GPC
            pltpu.sync_copy(src2d.at[pl.ds(row0, GPC), :], idx)
            hs = [pltpu.async_copy(tab.at[idx.at[j]],
                                   rows.at[pl.ds(j * GL, GL), :], sem)
                  for j in range(GPC)]
            for h in hs:
                h.wait()
            pltpu.sync_copy(rows, out.at[pl.ds(row0 * GL, CHUNK), :])
            return 0

        lax.fori_loop(0, CPT_P, chunk, 0)

    @pl.when(c == 0)
    def _():
        run(sh_tab, xsh, out_sh)

    @pl.when(c == 1)
    def _():
        run(co_tab, xco, out_co)


def _emb(sh_tab, co_tab, xsh2d, xco2d):
    return pl.kernel(
        _emb_body,
        out_type=[jax.ShapeDtypeStruct((NPOOL, EMB), F32),
                  jax.ShapeDtypeStruct((NPOOL, EMB), F32)],
        mesh=_MESH,
        compiler_params=_SC_PARAMS,
        scratch_types=[
            pltpu.VMEM((GPC, GL), jnp.int32),
            pltpu.VMEM((CHUNK, EMB), F32),
            pltpu.SemaphoreType.DMA,
        ],
    )(sh_tab, co_tab, xsh2d, xco2d)


def _agg_body(src0, dst0, src1, dst1, src2, dst2, hq0, hq1, hq2, hq3,
              o0_lo, o1_lo, o2_lo, o0_hi, o1_hi, o2_hi,
              sidxA, didxA, sidxB, didxB, rowsA, rowsB, zbuf, acc,
              semGA, semGB, semSA, semSB):
    c = lax.axis_index("c")
    s = lax.axis_index("s")
    _fill(zbuf, ZR, QW, jnp.zeros((16,), F32))
    srcs = (src0, src1, src2)
    dsts = (dst0, dst1, dst2)

    def run(hA, hB, outs):
        for r in range(3):
            for hq, col in ((hA, 0), (hB, QW)):

                def stage(ch, sidx, didx):
                    row0 = (s * CPT_E + ch) * GPC
                    pltpu.sync_copy(srcs[r].at[pl.ds(row0, GPC), :], sidx)
                    pltpu.sync_copy(dsts[r].at[pl.ds(row0, GPC), :], didx)

                def issue_g(sidx, rows, sem):
                    for j in range(GPC):
                        pltpu.async_copy(hq.at[sidx.at[j]],
                                         rows.at[pl.ds(j * GL, GL), :], sem)

                def wait_g(sidx, rows, sem):
                    for j in range(GPC):
                        pltpu.make_async_copy(
                            hq.at[sidx.at[j]],
                            rows.at[pl.ds(j * GL, GL), :], sem).wait()

                def issue_s(didx, rows, sem):
                    for j in range(GPC):
                        pltpu.async_copy(rows.at[pl.ds(j * GL, GL), :],
                                         acc.at[didx.at[j]], sem, add=True)

                def wait_s(didx, rows, sem):
                    for j in range(GPC):
                        pltpu.make_async_copy(
                            rows.at[pl.ds(j * GL, GL), :],
                            acc.at[didx.at[j]], sem).wait()

                for k in range(5):
                    pltpu.sync_copy(
                        zbuf, acc.at[pl.ds(s * NPT + k * ZR, ZR), :])
                plsc.subcore_barrier()

                stage(0, sidxA, didxA)
                issue_g(sidxA, rowsA, semGA)

                def pair(i, _):
                    stage(2 * i + 1, sidxB, didxB)
                    wait_g(sidxA, rowsA, semGA)
                    issue_s(didxA, rowsA, semSA)
                    issue_g(sidxB, rowsB, semGB)
                    wait_s(didxA, rowsA, semSA)

                    @pl.when(2 * i + 2 < CPT_E)
                    def _():
                        stage(2 * i + 2, sidxA, didxA)
                        issue_g(sidxA, rowsA, semGA)

                    wait_g(sidxB, rowsB, semGB)
                    issue_s(didxB, rowsB, semSB)
                    wait_s(didxB, rowsB, semSB)
                    return 0

                lax.fori_loop(0, CPT_E // 2, pair, 0)
                plsc.subcore_barrier()
                pltpu.sync_copy(acc.at[pl.ds(s * NPT, NPT), :],
                                outs[r].at[pl.ds(s * NPT, NPT),
                                           pl.ds(col, QW)])
                plsc.subcore_barrier()

    @pl.when(c == 0)
    def _():
        run(hq0, hq1, (o0_lo, o1_lo, o2_lo))

    @pl.when(c == 1)
    def _():
        run(hq2, hq3, (o0_hi, o1_hi, o2_hi))


def _agg(edges2d, hqs):
    out = jax.ShapeDtypeStruct((N, EMB), F32)
    return pl.kernel(
        _agg_body,
        out_type=[out] * 6,
        mesh=_MESH,
        compiler_params=_SC_PARAMS,
        scratch_types=[
            pltpu.VMEM((GPC, GL), jnp.int32),
            pltpu.VMEM((GPC, GL), jnp.int32),
            pltpu.VMEM((GPC, GL), jnp.int32),
            pltpu.VMEM((GPC, GL), jnp.int32),
            pltpu.VMEM((CHUNK, QW), F32),
            pltpu.VMEM((CHUNK, QW), F32),
            pltpu.VMEM((ZR, QW), F32),
            pltpu.VMEM_SHARED((N_ACC, QW), F32),
            pltpu.SemaphoreType.DMA,
            pltpu.SemaphoreType.DMA,
            pltpu.SemaphoreType.DMA,
            pltpu.SemaphoreType.DMA,
        ],
    )(edges2d[0][0], edges2d[0][1], edges2d[1][0], edges2d[1][1],
      edges2d[2][0], edges2d[2][1], *hqs)


NH = 50176
HPT = NH // 16


def _cnt_body(dst0, dst1, dst2, dstp, o0, o1, o2, og,
              didx, ones1, zb1, hist, sem2):
    c = lax.axis_index("c")
    s = lax.axis_index("s")
    wid = c * 16 + s
    one = jnp.ones((16,), F32)
    zero = jnp.zeros((16,), F32)

    def fill1(i, _):
        ones1[pl.ds(i * 16, 16)] = one
        return 0

    lax.fori_loop(0, GL // 16, fill1, 0)

    def fillz(i, _):
        zb1[pl.ds(i * 16, 16)] = zero
        return 0

    lax.fori_loop(0, HPT // 16, fillz, 0)

    dsts = (dst0, dst1, dst2)
    outs = (o0, o1, o2)

    def add_chunk(dst2d, ch_abs):
        row0 = ch_abs * GPC
        pltpu.sync_copy(dst2d.at[pl.ds(row0, GPC), :], didx)
        hs = [pltpu.async_copy(ones1, hist.at[didx.at[j]], sem2, add=True)
              for j in range(GPC)]
        for h in hs:
            h.wait()

    n_ch = 25
    base_ch = wid * 25
    for r in range(3):
        pltpu.sync_copy(zb1, hist.at[pl.ds(s * HPT, HPT)])
        plsc.subcore_barrier()

        def chunk(i, _):
            add_chunk(dsts[r], base_ch + i)
            return 0

        lax.fori_loop(0, n_ch, chunk, 0)
        plsc.subcore_barrier()
        pltpu.sync_copy(hist.at[pl.ds(s * HPT, HPT)],
                        outs[r].at[c, pl.ds(s * HPT, HPT)])
        plsc.subcore_barrier()

    pltpu.sync_copy(zb1.at[pl.ds(0, 128)], hist.at[pl.ds(s * 128, 128)])
    plsc.subcore_barrier()

    def chunkb(i, _):
        add_chunk(dstp, wid * 2 + i)
        return 0

    lax.fori_loop(0, 2, chunkb, 0)
    plsc.subcore_barrier()
    pltpu.sync_copy(hist.at[pl.ds(s * 64, 64)], og.at[c, pl.ds(s * 64, 64)])


def _cnt(dst2d_rels, dstp2d):
    return pl.kernel(
        _cnt_body,
        out_type=[jax.ShapeDtypeStruct((2, NH), F32)] * 3
        + [jax.ShapeDtypeStruct((2, NG), F32)],
        mesh=_MESH,
        compiler_params=_SC_PARAMS,
        scratch_types=[
            pltpu.VMEM((GPC, GL), jnp.int32),
            pltpu.VMEM((GL,), F32),
            pltpu.VMEM((HPT,), F32),
            pltpu.VMEM_SHARED((NH,), F32),
            pltpu.SemaphoreType.DMA,
        ],
    )(dst2d_rels[0], dst2d_rels[1], dst2d_rels[2], dstp2d)


def _pool_body(srcp, dstp, hq0, hq1, hq2, hq3, o_lo, o_hi,
               sidx, didx, rows, zbuf, acc, sem, sem2):
    c = lax.axis_index("c")
    s = lax.axis_index("s")
    _fill(zbuf, 65, QW, jnp.zeros((16,), F32))

    def run(hA, hB, out):
        for hq, col in ((hA, 0), (hB, QW)):
            pltpu.sync_copy(zbuf, acc.at[pl.ds(s * 65, 65), :])
            plsc.subcore_barrier()

            def chunk(ch, _):
                row0 = (s * CPT_P + ch) * GPC
                pltpu.sync_copy(srcp.at[pl.ds(row0, GPC), :], sidx)
                pltpu.sync_copy(dstp.at[pl.ds(row0, GPC), :], didx)
                hs = [pltpu.async_copy(hq.at[sidx.at[j]],
                                       rows.at[pl.ds(j * GL, GL), :], sem)
                      for j in range(GPC)]
                for h in hs:
                    h.wait()
                hs2 = [pltpu.async_copy(rows.at[pl.ds(j * GL, GL), :],
                                        acc.at[didx.at[j]], sem2, add=True)
                       for j in range(GPC)]
                for h in hs2:
                    h.wait()
                return 0

            lax.fori_loop(0, CPT_P, chunk, 0)
            plsc.subcore_barrier()
            pltpu.sync_copy(acc.at[pl.ds(s * 64, 64), :],
                            out.at[pl.ds(s * 64, 64), pl.ds(col, QW)])
            plsc.subcore_barrier()

    @pl.when(c == 0)
    def _():
        run(hq0, hq1, o_lo)

    @pl.when(c == 1)
    def _():
        run(hq2, hq3, o_hi)


def _pool(srcp2d, dstp2d, hqs):
    out = jax.ShapeDtypeStruct((NG, EMB), F32)
    return pl.kernel(
        _pool_body,
        out_type=[out, out],
        mesh=_MESH,
        compiler_params=_SC_PARAMS,
        scratch_types=[
            pltpu.VMEM((GPC, GL), jnp.int32),
            pltpu.VMEM((GPC, GL), jnp.int32),
            pltpu.VMEM((CHUNK, QW), F32),
            pltpu.VMEM((65, QW), F32),
            pltpu.VMEM_SHARED((NG_ACC, QW), F32),
            pltpu.SemaphoreType.DMA,
            pltpu.SemaphoreType.DMA,
        ],
    )(srcp2d, dstp2d, *hqs)


def _split_q(y, outs):
    for q in range(4):
        outs[q][...] = y[:, q * QW:(q + 1) * QW]


def _mlp0_body(sh, co, w, b, oq0, oq1, oq2, oq3):
    x = jnp.concatenate([sh[...], co[...]], axis=1)
    y = jax.nn.relu(
        jnp.dot(x, w[...], preferred_element_type=F32) + b[...])
    _split_q(y, (oq0, oq1, oq2, oq3))


def _mlp0(sh, co, wT, b):
    grid = N // BN
    half = pl.BlockSpec((BN, EMB), lambda i: (i, 0))
    quart = pl.BlockSpec((BN, QW), lambda i: (i, 0))
    return pl.pallas_call(
        _mlp0_body,
        grid=(grid,),
        in_specs=[half, half,
                  pl.BlockSpec((2 * EMB, HID), lambda i: (0, 0)),
                  pl.BlockSpec((1, HID), lambda i: (0, 0))],
        out_specs=[quart] * 4,
        out_shape=[jax.ShapeDtypeStruct((N, QW), F32)] * 4,
    )(sh, co, wT, b.reshape(1, HID))


def _layer_body(a0l, a0h, a1l, a1h, a2l, a2h, c0, c1, c2,
                hq0, hq1, hq2, hq3, w, b, oq0, oq1, oq2, oq3):
    r0 = 1.0 / jnp.maximum(c0[...], 1.0)
    r1 = 1.0 / jnp.maximum(c1[...], 1.0)
    r2 = 1.0 / jnp.maximum(c2[...], 1.0)
    x = jnp.concatenate(
        [a0l[...] * r0, a0h[...] * r0,
         a1l[...] * r1, a1h[...] * r1,
         a2l[...] * r2, a2h[...] * r2,
         hq0[...], hq1[...], hq2[...], hq3[...]], axis=1)
    y = jax.nn.relu(
        jnp.dot(x, w[...], preferred_element_type=F32) + b[...])
    _split_q(y, (oq0, oq1, oq2, oq3))


def _layer(aggs, cnts, hqs, wT_cat, b_sum):
    grid = N // BN
    half = pl.BlockSpec((BN, EMB), lambda i: (i, 0))
    quart = pl.BlockSpec((BN, QW), lambda i: (i, 0))
    col = pl.BlockSpec((BN, 1), lambda i: (i, 0))
    return pl.pallas_call(
        _layer_body,
        grid=(grid,),
        in_specs=[half] * 6 + [col] * 3 + [quart] * 4 + [
            pl.BlockSpec((4 * HID, HID), lambda i: (0, 0)),
            pl.BlockSpec((1, HID), lambda i: (0, 0))],
        out_specs=[quart] * 4,
        out_shape=[jax.ShapeDtypeStruct((N, QW), F32)] * 4,
    )(*aggs, *cnts, *hqs, wT_cat, b_sum.reshape(1, HID))


def _readout_body(gl_, gh, c, w, b, out):
    rc = 1.0 / jnp.maximum(c[...], 1.0)
    pooled = jnp.concatenate([gl_[...] * rc, gh[...] * rc], axis=1)
    out[...] = jnp.dot(pooled, w[...], preferred_element_type=F32) + b[...]


def _readout(gp_lo, gp_hi, gcnt, woutT, bout):
    return pl.pallas_call(
        _readout_body,
        in_specs=[pl.BlockSpec((NG, EMB), lambda: (0, 0)),
                  pl.BlockSpec((NG, EMB), lambda: (0, 0)),
                  pl.BlockSpec((NG, 1), lambda: (0, 0)),
                  pl.BlockSpec((HID, NCLS), lambda: (0, 0)),
                  pl.BlockSpec((1, NCLS), lambda: (0, 0))],
        out_specs=pl.BlockSpec((NG, NCLS), lambda: (0, 0)),
        out_shape=jax.ShapeDtypeStruct((NG, NCLS), F32),
    )(gp_lo, gp_hi, gcnt, woutT, bout.reshape(1, NCLS))


def kernel(x, edge_index_seq, edge_index_shape, edge_index_color, batch,
           shape_emb, color_emb, W0, b0,
           rel1_Wl, rel1_bl, rel1_Wr,
           rel2_Wl, rel2_bl, rel2_Wr,
           Wout, bout):
    eis = (edge_index_seq, edge_index_shape, edge_index_color)

    pad_e = E_PAD - E
    pad_src = (jnp.arange(pad_e, dtype=jnp.int32) % 1024).astype(eis[0].dtype)
    pad_dst = (N + jnp.arange(pad_e, dtype=jnp.int32) % 16).astype(eis[0].dtype)
    edges2d = []
    for ei in eis:
        src = jnp.concatenate([ei[0], pad_src])
        dst = jnp.concatenate([ei[1], pad_dst])
        edges2d.append((src.reshape(EROWS, GL).astype(jnp.int32),
                        dst.reshape(EROWS, GL).astype(jnp.int32)))

    pad_n = NPOOL - N
    xi = x.astype(jnp.int32)
    xsh2d = jnp.concatenate([xi[:, 0], jnp.zeros((pad_n,), jnp.int32)]
                            ).reshape(PROWS, GL)
    xco2d = jnp.concatenate([xi[:, 1], jnp.zeros((pad_n,), jnp.int32)]
                            ).reshape(PROWS, GL)
    srcp2d = jnp.concatenate([jnp.arange(N, dtype=jnp.int32),
                              jnp.zeros((pad_n,), jnp.int32)]
                             ).reshape(PROWS, GL)
    dstp2d = jnp.concatenate([batch.astype(jnp.int32),
                              jnp.full((pad_n,), NG, jnp.int32)]
                             ).reshape(PROWS, GL)

    esh, eco = _emb(shape_emb, color_emb, xsh2d, xco2d)
    cnt_parts = _cnt([e[1] for e in edges2d], dstp2d)
    cnts = [(p[0, :N] + p[1, :N]).reshape(N, 1) for p in cnt_parts[:3]]
    gcnt = (cnt_parts[3][0] + cnt_parts[3][1]).reshape(NG, 1)

    hqs = _mlp0(esh[:N], eco[:N], W0.T, b0)

    for Wl, bl, Wr in ((rel1_Wl, rel1_bl, rel1_Wr),
                       (rel2_Wl, rel2_bl, rel2_Wr)):
        aggs = _agg(edges2d, hqs)
        aggs = (aggs[0], aggs[3], aggs[1], aggs[4], aggs[2], aggs[5])
        wT_cat = jnp.concatenate(
            [Wl[0].T, Wl[1].T, Wl[2].T, (Wr[0] + Wr[1] + Wr[2]).T], axis=0)
        hqs = _layer(aggs, cnts, hqs, wT_cat, bl[0] + bl[1] + bl[2])

    gp_lo, gp_hi = _pool(srcp2d, dstp2d, hqs)
    return _readout(gp_lo, gp_hi, gcnt, Wout.T, bout)


# speedup vs baseline: 7.5963x; 1.1773x over previous
import jax
import jax.numpy as jnp
from jax import lax
from jax.experimental import pallas as pl
from jax.experimental.pallas import tpu as pltpu
from jax.experimental.pallas import tpu_sc as plsc

N = 50000
E = 800000
EMB = 32
HID = 64
NCLS = 2
NG = 1024

F32 = jnp.float32
QW = 16

GL = 128
GPC = 8
CHUNK = GL * GPC
CPT_E = 50
E_PAD = 16 * CPT_E * CHUNK
EROWS = E_PAD // GL

NPOOL = 65536
CPT_P = NPOOL // (16 * CHUNK)
PROWS = NPOOL // GL

NPT = N // 16
ZR = 625
N_ACC = N + 16
NG_ACC = 1040

BN = 2000
CCOL = 64

_MESH = plsc.VectorSubcoreMesh(core_axis_name="c", subcore_axis_name="s")
_SC_PARAMS = pltpu.CompilerParams(use_tc_tiling_on_sc=False)


def _fill(buf, nrows, ncols, vec):
    def body(i, _):
        for j in range(ncols // 16):
            buf[i, pl.ds(j * 16, 16)] = vec
        return 0

    lax.fori_loop(0, nrows, body, 0)


def _emb_body(sh_tab, co_tab, xsh, xco, out, idx, rows, sem):
    c = lax.axis_index("c")
    s = lax.axis_index("s")

    def run(tab, src2d, col):
        def chunk(ch, _):
            row0 = (s * CPT_P + ch) * GPC
            pltpu.sync_copy(src2d.at[pl.ds(row0, GPC), :], idx)
            hs = [pltpu.async_copy(tab.at[idx.at[j]],
                                   rows.at[pl.ds(j * GL, GL), :], sem)
                  for j in range(GPC)]
            for h in hs:
                h.wait()
            pltpu.sync_copy(rows, out.at[pl.ds(row0 * GL, CHUNK),
                                         pl.ds(col, EMB)])
            return 0

        lax.fori_loop(0, CPT_P, chunk, 0)

    @pl.when(c == 0)
    def _():
        run(sh_tab, xsh, 0)

    @pl.when(c == 1)
    def _():
        run(co_tab, xco, EMB)


def _emb(sh_tab, co_tab, xsh2d, xco2d):
    return pl.kernel(
        _emb_body,
        out_type=jax.ShapeDtypeStruct((NPOOL, 128), F32),
        mesh=_MESH,
        compiler_params=_SC_PARAMS,
        scratch_types=[
            pltpu.VMEM((GPC, GL), jnp.int32),
            pltpu.VMEM((CHUNK, EMB), F32),
            pltpu.SemaphoreType.DMA,
        ],
    )(sh_tab, co_tab, xsh2d, xco2d)


def _agg_body(src0, dst0, src1, dst1, src2, dst2, dstp, h8,
              o0, o1, o2, og,
              sidxA, didxA, sidxB, didxB, rowsA, rowsB, zbuf, acc,
              semGA, semGB, semSA, semSB, with_counts):
    c = lax.axis_index("c")
    s = lax.axis_index("s")
    zero16 = jnp.zeros((16,), F32)
    _fill(zbuf, ZR, QW, zero16)
    e0 = jnp.where(lax.iota(jnp.int32, 16) == 0, 1.0, 0.0)
    srcs = (src0, src1, src2)
    dsts = (dst0, dst1, dst2)
    outs = (o0, o1, o2)

    def zero_acc():
        for k in range(5):
            pltpu.sync_copy(zbuf, acc.at[pl.ds(s * NPT + k * ZR, ZR), :])
        plsc.subcore_barrier()

    def drain(out, col):
        plsc.subcore_barrier()
        pltpu.sync_copy(acc.at[pl.ds(s * NPT, NPT), :],
                        out.at[pl.ds(s * NPT, NPT), pl.ds(col, QW)])
        plsc.subcore_barrier()

    def run(base_col):
        for r in range(3):
            for col in (base_col, base_col + QW):
                qq = col // QW

                def stage(ch, sidx, didx):
                    row0 = (s * CPT_E + ch) * GPC
                    pltpu.sync_copy(srcs[r].at[pl.ds(row0, GPC), :], sidx)
                    pltpu.sync_copy(dsts[r].at[pl.ds(row0, GPC), :], didx)
                    def xf(i, _):
                        for k in range(GL // 16):
                            v = sidx[i, pl.ds(k * 16, 16)]
                            sidx[i, pl.ds(k * 16, 16)] = v * 8 + qq
                        return 0
                    lax.fori_loop(0, GPC, xf, 0)

                def issue_g(sidx, rows, sem):
                    for j in range(GPC):
                        pltpu.async_copy(
                            h8.at[sidx.at[j]],
                            rows.at[pl.ds(j * GL, GL), :], sem)

                def wait_g(sidx, rows, sem):
                    for j in range(GPC):
                        pltpu.make_async_copy(
                            h8.at[sidx.at[j]],
                            rows.at[pl.ds(j * GL, GL), :], sem).wait()

                def issue_s(didx, rows, sem):
                    for j in range(GPC):
                        pltpu.async_copy(rows.at[pl.ds(j * GL, GL), :],
                                         acc.at[didx.at[j]], sem, add=True)

                def wait_s(didx, rows, sem):
                    for j in range(GPC):
                        pltpu.make_async_copy(
                            rows.at[pl.ds(j * GL, GL), :],
                            acc.at[didx.at[j]], sem).wait()

                zero_acc()
                stage(0, sidxA, didxA)
                issue_g(sidxA, rowsA, semGA)

                def pair(i, _):
                    stage(2 * i + 1, sidxB, didxB)
                    wait_g(sidxA, rowsA, semGA)
                    issue_s(didxA, rowsA, semSA)
                    issue_g(sidxB, rowsB, semGB)
                    wait_s(didxA, rowsA, semSA)

                    @pl.when(2 * i + 2 < CPT_E)
                    def _():
                        stage(2 * i + 2, sidxA, didxA)
                        issue_g(sidxA, rowsA, semGA)

                    wait_g(sidxB, rowsB, semGB)
                    issue_s(didxB, rowsB, semSB)
                    wait_s(didxB, rowsB, semSB)
                    return 0

                lax.fori_loop(0, CPT_E // 2, pair, 0)
                drain(outs[r], col)

    def cnt_run(dst2d, out):
        _fill(rowsA, GL, QW, e0)

        def stage(ch, didx):
            row0 = (s * CPT_E + ch) * GPC
            pltpu.sync_copy(dst2d.at[pl.ds(row0, GPC), :], didx)

        def issue_s(didx, sem):
            for j in range(GPC):
                pltpu.async_copy(rowsA.at[pl.ds(0, GL), :],
                                 acc.at[didx.at[j]], sem, add=True)

        def wait_s(didx, sem):
            for j in range(GPC):
                pltpu.make_async_copy(rowsA.at[pl.ds(0, GL), :],
                                      acc.at[didx.at[j]], sem).wait()

        zero_acc()
        stage(0, didxA)
        issue_s(didxA, semSA)

        def pair(i, _):
            stage(2 * i + 1, didxB)
            wait_s(didxA, semSA)
            issue_s(didxB, semSB)

            @pl.when(2 * i + 2 < CPT_E)
            def _():
                stage(2 * i + 2, didxA)
                issue_s(didxA, semSA)

            wait_s(didxB, semSB)
            return 0

        lax.fori_loop(0, CPT_E // 2, pair, 0)
        drain(out, CCOL)

    def cnt_batch():
        _fill(rowsA, GL, QW, e0)
        pltpu.sync_copy(zbuf.at[pl.ds(0, 65), :],
                        acc.at[pl.ds(s * 65, 65), :])
        plsc.subcore_barrier()

        def chunk(ch, _):
            row0 = (s * CPT_P + ch) * GPC
            pltpu.sync_copy(dstp.at[pl.ds(row0, GPC), :], didxA)
            hs = [pltpu.async_copy(rowsA.at[pl.ds(0, GL), :],
                                   acc.at[didxA.at[j]], semSA, add=True)
                  for j in range(GPC)]
            for h in hs:
                h.wait()
            return 0

        lax.fori_loop(0, CPT_P, chunk, 0)
        plsc.subcore_barrier()
        pltpu.sync_copy(acc.at[pl.ds(s * 64, 64), :],
                        og.at[pl.ds(s * 64, 64), pl.ds(CCOL, QW)])

    @pl.when(c == 0)
    def _():
        run(0)
        if with_counts:
            cnt_run(dst0, o0)
            cnt_run(dst1, o1)

    @pl.when(c == 1)
    def _():
        run(2 * QW)
        if with_counts:
            cnt_run(dst2, o2)
            cnt_batch()


def _agg(edges2d, dstp2d, h8, with_counts):
    def body(*refs):
        _agg_body(*refs, with_counts=with_counts)

    return pl.kernel(
        body,
        out_type=[jax.ShapeDtypeStruct((N, 128), F32)] * 3
        + [jax.ShapeDtypeStruct((NG, 128), F32)],
        mesh=_MESH,
        compiler_params=_SC_PARAMS,
        scratch_types=[
            pltpu.VMEM((GPC, GL), jnp.int32),
            pltpu.VMEM((GPC, GL), jnp.int32),
            pltpu.VMEM((GPC, GL), jnp.int32),
            pltpu.VMEM((GPC, GL), jnp.int32),
            pltpu.VMEM((CHUNK, QW), F32),
            pltpu.VMEM((CHUNK, QW), F32),
            pltpu.VMEM((ZR, QW), F32),
            pltpu.VMEM_SHARED((N_ACC, QW), F32),
            pltpu.SemaphoreType.DMA,
            pltpu.SemaphoreType.DMA,
            pltpu.SemaphoreType.DMA,
            pltpu.SemaphoreType.DMA,
        ],
    )(edges2d[0][0], edges2d[0][1], edges2d[1][0], edges2d[1][1],
      edges2d[2][0], edges2d[2][1], dstp2d, h8)


def _pool_body(srcp, dstp, h8, out,
               sidx, didx, rows, zbuf, acc, sem, sem2):
    c = lax.axis_index("c")
    s = lax.axis_index("s")
    _fill(zbuf, 65, QW, jnp.zeros((16,), F32))

    def run(base_col):
        for q in range(2):
            col = base_col + q * QW
            qq = col // QW
            pltpu.sync_copy(zbuf, acc.at[pl.ds(s * 65, 65), :])
            plsc.subcore_barrier()

            def chunk(ch, _):
                row0 = (s * CPT_P + ch) * GPC
                pltpu.sync_copy(srcp.at[pl.ds(row0, GPC), :], sidx)
                pltpu.sync_copy(dstp.at[pl.ds(row0, GPC), :], didx)

                def xf(i, _):
                    for k in range(GL // 16):
                        v = sidx[i, pl.ds(k * 16, 16)]
                        sidx[i, pl.ds(k * 16, 16)] = v * 8 + qq
                    return 0
                lax.fori_loop(0, GPC, xf, 0)
                hs = [pltpu.async_copy(
                    h8.at[sidx.at[j]],
                    rows.at[pl.ds(j * GL, GL), :], sem)
                    for j in range(GPC)]
                for h in hs:
                    h.wait()
                hs2 = [pltpu.async_copy(rows.at[pl.ds(j * GL, GL), :],
                                        acc.at[didx.at[j]], sem2, add=True)
                       for j in range(GPC)]
                for h in hs2:
                    h.wait()
                return 0

            lax.fori_loop(0, CPT_P, chunk, 0)
            plsc.subcore_barrier()
            pltpu.sync_copy(acc.at[pl.ds(s * 64, 64), :],
                            out.at[pl.ds(s * 64, 64), pl.ds(col, QW)])
            plsc.subcore_barrier()

    @pl.when(c == 0)
    def _():
        run(0)

    @pl.when(c == 1)
    def _():
        run(2 * QW)


def _pool(srcp2d, dstp2d, h8):
    return pl.kernel(
        _pool_body,
        out_type=jax.ShapeDtypeStruct((NG, 128), F32),
        mesh=_MESH,
        compiler_params=_SC_PARAMS,
        scratch_types=[
            pltpu.VMEM((GPC, GL), jnp.int32),
            pltpu.VMEM((GPC, GL), jnp.int32),
            pltpu.VMEM((CHUNK, QW), F32),
            pltpu.VMEM((65, QW), F32),
            pltpu.VMEM_SHARED((NG_ACC, QW), F32),
            pltpu.SemaphoreType.DMA,
            pltpu.SemaphoreType.DMA,
        ],
    )(srcp2d, dstp2d, h8)


def _pad128(y):
    return jnp.concatenate(
        [y, jnp.zeros((y.shape[0], 128 - y.shape[1]), F32)], axis=1)


def _mlp0_body(x, w, b, out):
    out[...] = _pad128(jax.nn.relu(
        jnp.dot(x[...][:, :HID], w[...], preferred_element_type=F32)
        + b[...]))


def _mlp0(emb128, wT, b):
    grid = N // BN
    return pl.pallas_call(
        _mlp0_body,
        grid=(grid,),
        in_specs=[pl.BlockSpec((BN, 128), lambda i: (i, 0)),
                  pl.BlockSpec((2 * EMB, HID), lambda i: (0, 0)),
                  pl.BlockSpec((1, HID), lambda i: (0, 0))],
        out_specs=pl.BlockSpec((BN, 128), lambda i: (i, 0)),
        out_shape=jax.ShapeDtypeStruct((N, 128), F32),
    )(emb128, wT, b.reshape(1, HID))


def _layer_body(a0, a1, a2, c0, c1, c2, h, w0, w1, w2, wr, b, out):
    y = jnp.dot(a0[...][:, :HID], w0[...], preferred_element_type=F32) \
        * (1.0 / jnp.maximum(c0[...][:, CCOL:CCOL + 1], 1.0))
    y += jnp.dot(a1[...][:, :HID], w1[...], preferred_element_type=F32) \
        * (1.0 / jnp.maximum(c1[...][:, CCOL:CCOL + 1], 1.0))
    y += jnp.dot(a2[...][:, :HID], w2[...], preferred_element_type=F32) \
        * (1.0 / jnp.maximum(c2[...][:, CCOL:CCOL + 1], 1.0))
    y += jnp.dot(h[...][:, :HID], wr[...], preferred_element_type=F32) \
        + b[...]
    out[...] = _pad128(jax.nn.relu(y))


def _layer(aggs, cnt_srcs, h128, wTs, wrT, b_sum):
    grid = N // BN
    feat = pl.BlockSpec((BN, 128), lambda i: (i, 0))
    wspec = pl.BlockSpec((HID, HID), lambda i: (0, 0))
    return pl.pallas_call(
        _layer_body,
        grid=(grid,),
        in_specs=[feat, feat, feat, feat, feat, feat, feat,
                  wspec, wspec, wspec, wspec,
                  pl.BlockSpec((1, HID), lambda i: (0, 0))],
        out_specs=feat,
        out_shape=jax.ShapeDtypeStruct((N, 128), F32),
    )(*aggs, *cnt_srcs, h128, *wTs, wrT, b_sum.reshape(1, HID))


def _readout_body(p, g, w, b, out):
    rc = 1.0 / jnp.maximum(g[...][:, CCOL:CCOL + 1], 1.0)
    pooled = p[...][:, :HID] * rc
    out[...] = jnp.dot(pooled, w[...], preferred_element_type=F32) + b[...]


def _readout(pool128, og, woutT, bout):
    return pl.pallas_call(
        _readout_body,
        in_specs=[pl.BlockSpec((NG, 128), lambda: (0, 0)),
                  pl.BlockSpec((NG, 128), lambda: (0, 0)),
                  pl.BlockSpec((HID, NCLS), lambda: (0, 0)),
                  pl.BlockSpec((1, NCLS), lambda: (0, 0))],
        out_specs=pl.BlockSpec((NG, NCLS), lambda: (0, 0)),
        out_shape=jax.ShapeDtypeStruct((NG, NCLS), F32),
    )(pool128, og, woutT, bout.reshape(1, NCLS))


def kernel(x, edge_index_seq, edge_index_shape, edge_index_color, batch,
           shape_emb, color_emb, W0, b0,
           rel1_Wl, rel1_bl, rel1_Wr,
           rel2_Wl, rel2_bl, rel2_Wr,
           Wout, bout):
    eis = (edge_index_seq, edge_index_shape, edge_index_color)

    pad_e = E_PAD - E
    pad_src = (jnp.arange(pad_e, dtype=jnp.int32) % 1024).astype(eis[0].dtype)
    pad_dst = (N + jnp.arange(pad_e, dtype=jnp.int32) % 16).astype(eis[0].dtype)
    edges2d = []
    for ei in eis:
        src = jnp.concatenate([ei[0], pad_src])
        dst = jnp.concatenate([ei[1], pad_dst])
        edges2d.append((src.reshape(EROWS, GL).astype(jnp.int32),
                        dst.reshape(EROWS, GL).astype(jnp.int32)))

    pad_n = NPOOL - N
    xi = x.astype(jnp.int32)
    xsh2d = jnp.concatenate([xi[:, 0], jnp.zeros((pad_n,), jnp.int32)]
                            ).reshape(PROWS, GL)
    xco2d = jnp.concatenate([xi[:, 1], jnp.zeros((pad_n,), jnp.int32)]
                            ).reshape(PROWS, GL)
    srcp2d = jnp.concatenate([jnp.arange(N, dtype=jnp.int32),
                              jnp.zeros((pad_n,), jnp.int32)]
                             ).reshape(PROWS, GL)
    dstp2d = jnp.concatenate([batch.astype(jnp.int32),
                              jnp.full((pad_n,), NG, jnp.int32)]
                             ).reshape(PROWS, GL)

    emb128 = _emb(shape_emb, color_emb, xsh2d, xco2d)
    h128 = _mlp0(emb128, W0.T, b0)

    a0, a1, a2, og = _agg(edges2d, dstp2d, h128.reshape(N * 8, QW),
                          with_counts=True)
    w1Ts = [rel1_Wl[i].T for i in range(3)]
    h128 = _layer((a0, a1, a2), (a0, a1, a2), h128, w1Ts,
                  (rel1_Wr[0] + rel1_Wr[1] + rel1_Wr[2]).T,
                  rel1_bl[0] + rel1_bl[1] + rel1_bl[2])

    b0_, b1_, b2_, _ = _agg(edges2d, dstp2d, h128.reshape(N * 8, QW),
                            with_counts=False)
    w2Ts = [rel2_Wl[i].T for i in range(3)]
    h128 = _layer((b0_, b1_, b2_), (a0, a1, a2), h128, w2Ts,
                  (rel2_Wr[0] + rel2_Wr[1] + rel2_Wr[2]).T,
                  rel2_bl[0] + rel2_bl[1] + rel2_bl[2])

    pool128 = _pool(srcp2d, dstp2d, h128.reshape(N * 8, QW))
    return _readout(pool128, og, Wout.T, bout)


# speedup vs baseline: 8.6407x; 1.1375x over previous
import jax
import jax.numpy as jnp
from jax import lax
from jax.experimental import pallas as pl
from jax.experimental.pallas import tpu as pltpu
from jax.experimental.pallas import tpu_sc as plsc

N = 50000
E = 800000
EMB = 32
HID = 64
NCLS = 2
NG = 1024

F32 = jnp.float32
QW = 16

GL = 128
GPC = 8
CHUNK = GL * GPC
CPT_E = 50
E_PAD = 16 * CPT_E * CHUNK
EROWS = E_PAD // GL

NPOOL = 65536
CPT_P = NPOOL // (16 * CHUNK)
PROWS = NPOOL // GL
H_ROWS = 53248
CPT_B = H_ROWS // (32 * GL)
BROWS = H_ROWS // GL

NPT = N // 16
ZR = 625
N_ACC = N + 16
NG_ACC = 1040

BN = 2000
CCOL = 64

_MESH = plsc.VectorSubcoreMesh(core_axis_name="c", subcore_axis_name="s")
_SC_PARAMS = pltpu.CompilerParams(use_tc_tiling_on_sc=False)


def _fill(buf, nrows, ncols, vec):
    def body(i, _):
        for j in range(ncols // 16):
            buf[i, pl.ds(j * 16, 16)] = vec
        return 0

    lax.fori_loop(0, nrows, body, 0)


def _emb_body(sh_tab, co_tab, xsh, xco, out, idx, rows, sem):
    c = lax.axis_index("c")
    s = lax.axis_index("s")

    def run(tab, src2d, col):
        def chunk(ch, _):
            row0 = (s * CPT_P + ch) * GPC
            pltpu.sync_copy(src2d.at[pl.ds(row0, GPC), :], idx)
            hs = [pltpu.async_copy(tab.at[idx.at[j]],
                                   rows.at[pl.ds(j * GL, GL), :], sem)
                  for j in range(GPC)]
            for h in hs:
                h.wait()
            pltpu.sync_copy(rows, out.at[pl.ds(row0 * GL, CHUNK),
                                         pl.ds(col, EMB)])
            return 0

        lax.fori_loop(0, CPT_P, chunk, 0)

    @pl.when(c == 0)
    def _():
        run(sh_tab, xsh, 0)

    @pl.when(c == 1)
    def _():
        run(co_tab, xco, EMB)


def _emb(sh_tab, co_tab, xsh2d, xco2d):
    return pl.kernel(
        _emb_body,
        out_type=jax.ShapeDtypeStruct((NPOOL, 128), F32),
        mesh=_MESH,
        compiler_params=_SC_PARAMS,
        scratch_types=[
            pltpu.VMEM((GPC, GL), jnp.int32),
            pltpu.VMEM((CHUNK, EMB), F32),
            pltpu.SemaphoreType.DMA,
        ],
    )(sh_tab, co_tab, xsh2d, xco2d)


def _agg_body(src0, dst0, src1, dst1, src2, dst2, dstp, h8,
              o0, o1, o2, og,
              sidxA, didxA, sidxB, didxB, rowsA, rowsB, zbuf, acc,
              semGA, semGB, semSA, semSB, with_counts):
    c = lax.axis_index("c")
    s = lax.axis_index("s")
    zero16 = jnp.zeros((16,), F32)
    _fill(zbuf, ZR, QW, zero16)
    e0 = jnp.where(lax.iota(jnp.int32, 16) == 0, 1.0, 0.0)
    srcs = (src0, src1, src2)
    dsts = (dst0, dst1, dst2)
    outs = (o0, o1, o2)

    def zero_acc():
        for k in range(5):
            pltpu.sync_copy(zbuf, acc.at[pl.ds(s * NPT + k * ZR, ZR), :])
        plsc.subcore_barrier()

    def drain(out, col):
        plsc.subcore_barrier()
        pltpu.sync_copy(acc.at[pl.ds(s * NPT, NPT), :],
                        out.at[pl.ds(s * NPT, NPT), pl.ds(col, QW)])
        plsc.subcore_barrier()

    def run(base_col):
        for r in range(3):
            for col in (base_col, base_col + QW):
                qq = col // QW

                def stage(ch, sidx, didx):
                    row0 = (s * CPT_E + ch) * GPC
                    pltpu.sync_copy(srcs[r].at[pl.ds(row0, GPC), :], sidx)
                    pltpu.sync_copy(dsts[r].at[pl.ds(row0, GPC), :], didx)
                    def xf(i, _):
                        for k in range(GL // 16):
                            v = sidx[i, pl.ds(k * 16, 16)]
                            sidx[i, pl.ds(k * 16, 16)] = v * 8 + qq
                        return 0
                    lax.fori_loop(0, GPC, xf, 0)

                def issue_g(sidx, rows, sem):
                    for j in range(GPC):
                        pltpu.async_copy(
                            h8.at[sidx.at[j]],
                            rows.at[pl.ds(j * GL, GL), :], sem)

                def wait_g(sidx, rows, sem):
                    for j in range(GPC):
                        pltpu.make_async_copy(
                            h8.at[sidx.at[j]],
                            rows.at[pl.ds(j * GL, GL), :], sem).wait()

                def issue_s(didx, rows, sem):
                    for j in range(GPC):
                        pltpu.async_copy(rows.at[pl.ds(j * GL, GL), :],
                                         acc.at[didx.at[j]], sem, add=True)

                def wait_s(didx, rows, sem):
                    for j in range(GPC):
                        pltpu.make_async_copy(
                            rows.at[pl.ds(j * GL, GL), :],
                            acc.at[didx.at[j]], sem).wait()

                zero_acc()
                stage(0, sidxA, didxA)
                issue_g(sidxA, rowsA, semGA)

                def pair(i, _):
                    stage(2 * i + 1, sidxB, didxB)
                    wait_g(sidxA, rowsA, semGA)
                    issue_s(didxA, rowsA, semSA)
                    issue_g(sidxB, rowsB, semGB)
                    wait_s(didxA, rowsA, semSA)

                    @pl.when(2 * i + 2 < CPT_E)
                    def _():
                        stage(2 * i + 2, sidxA, didxA)
                        issue_g(sidxA, rowsA, semGA)

                    wait_g(sidxB, rowsB, semGB)
                    issue_s(didxB, rowsB, semSB)
                    wait_s(didxB, rowsB, semSB)
                    return 0

                lax.fori_loop(0, CPT_E // 2, pair, 0)
                drain(outs[r], col)

    def cnt_run(dst2d, out):
        _fill(rowsA, GL, QW, e0)

        def stage(ch, didx):
            row0 = (s * CPT_E + ch) * GPC
            pltpu.sync_copy(dst2d.at[pl.ds(row0, GPC), :], didx)

        def issue_s(didx, sem):
            for j in range(GPC):
                pltpu.async_copy(rowsA.at[pl.ds(0, GL), :],
                                 acc.at[didx.at[j]], sem, add=True)

        def wait_s(didx, sem):
            for j in range(GPC):
                pltpu.make_async_copy(rowsA.at[pl.ds(0, GL), :],
                                      acc.at[didx.at[j]], sem).wait()

        zero_acc()
        stage(0, didxA)
        issue_s(didxA, semSA)

        def pair(i, _):
            stage(2 * i + 1, didxB)
            wait_s(didxA, semSA)
            issue_s(didxB, semSB)

            @pl.when(2 * i + 2 < CPT_E)
            def _():
                stage(2 * i + 2, didxA)
                issue_s(didxA, semSA)

            wait_s(didxB, semSB)
            return 0

        lax.fori_loop(0, CPT_E // 2, pair, 0)
        drain(out, CCOL)

    def cnt_batch():
        _fill(rowsA, GL, QW, e0)
        pltpu.sync_copy(zbuf.at[pl.ds(0, 65), :],
                        acc.at[pl.ds(s * 65, 65), :])
        plsc.subcore_barrier()

        def chunk(i, _):
            pltpu.sync_copy(dstp.at[pl.ds(s * 26 + 2 * i, 2), :],
                            didxA.at[pl.ds(0, 2), :])
            for j in range(2):
                pltpu.async_copy(rowsA.at[pl.ds(0, GL), :],
                                 acc.at[didxA.at[j]], semSA,
                                 add=True).wait()
            return 0

        lax.fori_loop(0, 13, chunk, 0)
        plsc.subcore_barrier()
        pltpu.sync_copy(acc.at[pl.ds(s * 64, 64), :],
                        og.at[pl.ds(s * 64, 64), pl.ds(CCOL, QW)])

    @pl.when(c == 0)
    def _():
        run(0)
        if with_counts:
            cnt_run(dst0, o0)
            cnt_run(dst1, o1)

    @pl.when(c == 1)
    def _():
        run(2 * QW)
        if with_counts:
            cnt_run(dst2, o2)
            cnt_batch()


def _agg(edges2d, dstp2d, h8, with_counts):
    def body(*refs):
        _agg_body(*refs, with_counts=with_counts)

    return pl.kernel(
        body,
        out_type=[jax.ShapeDtypeStruct((N, 128), F32)] * 3
        + [jax.ShapeDtypeStruct((NG, 128), F32)],
        mesh=_MESH,
        compiler_params=_SC_PARAMS,
        scratch_types=[
            pltpu.VMEM((GPC, GL), jnp.int32),
            pltpu.VMEM((GPC, GL), jnp.int32),
            pltpu.VMEM((GPC, GL), jnp.int32),
            pltpu.VMEM((GPC, GL), jnp.int32),
            pltpu.VMEM((CHUNK, QW), F32),
            pltpu.VMEM((CHUNK, QW), F32),
            pltpu.VMEM((ZR, QW), F32),
            pltpu.VMEM_SHARED((N_ACC, QW), F32),
            pltpu.SemaphoreType.DMA,
            pltpu.SemaphoreType.DMA,
            pltpu.SemaphoreType.DMA,
            pltpu.SemaphoreType.DMA,
        ],
    )(edges2d[0][0], edges2d[0][1], edges2d[1][0], edges2d[1][1],
      edges2d[2][0], edges2d[2][1], dstp2d, h8)


def _pool_body(dstp, h128, o0, o1, didx, rows, zbuf, acc, sem, sem2):
    c = lax.axis_index("c")
    s = lax.axis_index("s")
    wid = c * 16 + s
    _fill(zbuf, 65, 128, jnp.zeros((16,), F32))
    pltpu.sync_copy(zbuf, acc.at[pl.ds(s * 65, 65), :])
    plsc.subcore_barrier()

    def chunk(i, _):
        ch = wid * CPT_B + i
        pltpu.async_copy(h128.at[pl.ds(ch * GL, GL), :], rows, sem)
        pltpu.sync_copy(dstp.at[pl.ds(ch, 1), :], didx)
        pltpu.make_async_copy(h128.at[pl.ds(ch * GL, GL), :], rows,
                              sem).wait()
        pltpu.async_copy(rows, acc.at[didx.at[0]], sem2, add=True).wait()
        return 0

    lax.fori_loop(0, CPT_B, chunk, 0)
    plsc.subcore_barrier()

    @pl.when(c == 0)
    def _():
        pltpu.sync_copy(acc.at[pl.ds(s * 64, 64), :],
                        o0.at[pl.ds(s * 64, 64), :])

    @pl.when(c == 1)
    def _():
        pltpu.sync_copy(acc.at[pl.ds(s * 64, 64), :],
                        o1.at[pl.ds(s * 64, 64), :])


def _pool(dstp2d, h128):
    out = jax.ShapeDtypeStruct((NG, 128), F32)
    return pl.kernel(
        _pool_body,
        out_type=[out, out],
        mesh=_MESH,
        compiler_params=_SC_PARAMS,
        scratch_types=[
            pltpu.VMEM((1, GL), jnp.int32),
            pltpu.VMEM((GL, 128), F32),
            pltpu.VMEM((65, 128), F32),
            pltpu.VMEM_SHARED((NG_ACC, 128), F32),
            pltpu.SemaphoreType.DMA,
            pltpu.SemaphoreType.DMA,
        ],
    )(dstp2d, h128)


def _pad128(y):
    return jnp.concatenate(
        [y, jnp.zeros((y.shape[0], 128 - y.shape[1]), F32)], axis=1)


def _mlp0_body(x, w, b, out):
    out[...] = _pad128(jax.nn.relu(
        jnp.dot(x[...][:, :HID], w[...], preferred_element_type=F32)
        + b[...]))


def _mlp0(emb128, wT, b):
    grid = N // BN
    return pl.pallas_call(
        _mlp0_body,
        grid=(grid,),
        in_specs=[pl.BlockSpec((BN, 128), lambda i: (i, 0)),
                  pl.BlockSpec((2 * EMB, HID), lambda i: (0, 0)),
                  pl.BlockSpec((1, HID), lambda i: (0, 0))],
        out_specs=pl.BlockSpec((BN, 128), lambda i: (i, 0)),
        out_shape=jax.ShapeDtypeStruct((H_ROWS, 128), F32),
    )(emb128, wT, b.reshape(1, HID))


def _layer_body(a0, a1, a2, c0, c1, c2, h, w0, w1, w2, wr, b, out):
    y = jnp.dot(a0[...][:, :HID], w0[...], preferred_element_type=F32) \
        * (1.0 / jnp.maximum(c0[...][:, CCOL:CCOL + 1], 1.0))
    y += jnp.dot(a1[...][:, :HID], w1[...], preferred_element_type=F32) \
        * (1.0 / jnp.maximum(c1[...][:, CCOL:CCOL + 1], 1.0))
    y += jnp.dot(a2[...][:, :HID], w2[...], preferred_element_type=F32) \
        * (1.0 / jnp.maximum(c2[...][:, CCOL:CCOL + 1], 1.0))
    y += jnp.dot(h[...][:, :HID], wr[...], preferred_element_type=F32) \
        + b[...]
    out[...] = _pad128(jax.nn.relu(y))


def _layer(aggs, cnt_srcs, h128, wTs, wrT, b_sum):
    grid = N // BN
    feat = pl.BlockSpec((BN, 128), lambda i: (i, 0))
    wspec = pl.BlockSpec((HID, HID), lambda i: (0, 0))
    return pl.pallas_call(
        _layer_body,
        grid=(grid,),
        in_specs=[feat, feat, feat, feat, feat, feat, feat,
                  wspec, wspec, wspec, wspec,
                  pl.BlockSpec((1, HID), lambda i: (0, 0))],
        out_specs=feat,
        out_shape=jax.ShapeDtypeStruct((H_ROWS, 128), F32),
    )(*aggs, *cnt_srcs, h128, *wTs, wrT, b_sum.reshape(1, HID))


def _readout_body(p0, p1, g, w, b, out):
    rc = 1.0 / jnp.maximum(g[...][:, CCOL:CCOL + 1], 1.0)
    pooled = (p0[...] + p1[...])[:, :HID] * rc
    out[...] = jnp.dot(pooled, w[...], preferred_element_type=F32) + b[...]


def _readout(p0, p1, og, woutT, bout):
    return pl.pallas_call(
        _readout_body,
        in_specs=[pl.BlockSpec((NG, 128), lambda: (0, 0)),
                  pl.BlockSpec((NG, 128), lambda: (0, 0)),
                  pl.BlockSpec((NG, 128), lambda: (0, 0)),
                  pl.BlockSpec((HID, NCLS), lambda: (0, 0)),
                  pl.BlockSpec((1, NCLS), lambda: (0, 0))],
        out_specs=pl.BlockSpec((NG, NCLS), lambda: (0, 0)),
        out_shape=jax.ShapeDtypeStruct((NG, NCLS), F32),
    )(p0, p1, og, woutT, bout.reshape(1, NCLS))


def kernel(x, edge_index_seq, edge_index_shape, edge_index_color, batch,
           shape_emb, color_emb, W0, b0,
           rel1_Wl, rel1_bl, rel1_Wr,
           rel2_Wl, rel2_bl, rel2_Wr,
           Wout, bout):
    eis = (edge_index_seq, edge_index_shape, edge_index_color)

    pad_e = E_PAD - E
    pad_src = (jnp.arange(pad_e, dtype=jnp.int32) % 1024).astype(eis[0].dtype)
    pad_dst = (N + jnp.arange(pad_e, dtype=jnp.int32) % 16).astype(eis[0].dtype)
    edges2d = []
    for ei in eis:
        src = jnp.concatenate([ei[0], pad_src])
        dst = jnp.concatenate([ei[1], pad_dst])
        edges2d.append((src.reshape(EROWS, GL).astype(jnp.int32),
                        dst.reshape(EROWS, GL).astype(jnp.int32)))

    pad_n = NPOOL - N
    xi = x.astype(jnp.int32)
    xsh2d = jnp.concatenate([xi[:, 0], jnp.zeros((pad_n,), jnp.int32)]
                            ).reshape(PROWS, GL)
    xco2d = jnp.concatenate([xi[:, 1], jnp.zeros((pad_n,), jnp.int32)]
                            ).reshape(PROWS, GL)
    dstp2d = jnp.concatenate([batch.astype(jnp.int32),
                              jnp.full((H_ROWS - N,), NG, jnp.int32)]
                             ).reshape(BROWS, GL)

    emb128 = _emb(shape_emb, color_emb, xsh2d, xco2d)
    h128 = _mlp0(emb128, W0.T, b0)

    a0, a1, a2, og = _agg(edges2d, dstp2d, h128.reshape(H_ROWS * 8, QW),
                          with_counts=True)
    w1Ts = [rel1_Wl[i].T for i in range(3)]
    h128 = _layer((a0, a1, a2), (a0, a1, a2), h128, w1Ts,
                  (rel1_Wr[0] + rel1_Wr[1] + rel1_Wr[2]).T,
                  rel1_bl[0] + rel1_bl[1] + rel1_bl[2])

    b0_, b1_, b2_, _ = _agg(edges2d, dstp2d, h128.reshape(H_ROWS * 8, QW),
                            with_counts=False)
    w2Ts = [rel2_Wl[i].T for i in range(3)]
    h128 = _layer((b0_, b1_, b2_), (a0, a1, a2), h128, w2Ts,
                  (rel2_Wr[0] + rel2_Wr[1] + rel2_Wr[2]).T,
                  rel2_bl[0] + rel2_bl[1] + rel2_bl[2])

    p0, p1 = _pool(dstp2d, h128)
    return _readout(p0, p1, og, Wout.T, bout)


# speedup vs baseline: 8.6412x; 1.0001x over previous
import jax
import jax.numpy as jnp
from jax import lax
from jax.experimental import pallas as pl
from jax.experimental.pallas import tpu as pltpu
from jax.experimental.pallas import tpu_sc as plsc

N = 50000
E = 800000
EMB = 32
HID = 64
NCLS = 2
NG = 1024

F32 = jnp.float32
QW = 16

GL = 128
GPC = 8
CHUNK = GL * GPC
CPT_E = 50
E_PAD = 16 * CPT_E * CHUNK
EROWS = E_PAD // GL

NPOOL = 65536
CPT_P = NPOOL // (16 * CHUNK)
PROWS = NPOOL // GL
H_ROWS = 53248
CPT_B = H_ROWS // (32 * GL)
BROWS = H_ROWS // GL

NPT = N // 16
ZR = 625
N_ACC = N + 16
NG_ACC = 1040

BN = 2000
CCOL = 64

_MESH = plsc.VectorSubcoreMesh(core_axis_name="c", subcore_axis_name="s")
_SC_PARAMS = pltpu.CompilerParams(use_tc_tiling_on_sc=False)


def _fill(buf, nrows, ncols, vec):
    def body(i, _):
        for j in range(ncols // 16):
            buf[i, pl.ds(j * 16, 16)] = vec
        return 0

    lax.fori_loop(0, nrows, body, 0)


def _emb_body(sh_tab, co_tab, xsh, xco, out,
              idxA, idxB, rowsA, rowsB, semGA, semGB, semWA, semWB):
    c = lax.axis_index("c")
    s = lax.axis_index("s")

    def run(tab, src2d, col):
        def stage(ch, idx):
            pltpu.sync_copy(src2d.at[pl.ds((s * CPT_P + ch) * GPC, GPC), :],
                            idx)

        def issue_g(idx, rows, sem):
            for j in range(GPC):
                pltpu.async_copy(tab.at[idx.at[j]],
                                 rows.at[pl.ds(j * GL, GL), :], sem)

        def wait_g(idx, rows, sem):
            for j in range(GPC):
                pltpu.make_async_copy(tab.at[idx.at[j]],
                                      rows.at[pl.ds(j * GL, GL), :],
                                      sem).wait()

        def out_ref(ch):
            return out.at[pl.ds((s * CPT_P + ch) * CHUNK, CHUNK),
                          pl.ds(col, EMB)]

        stage(0, idxA)
        issue_g(idxA, rowsA, semGA)

        def pair(i, _):
            stage(2 * i + 1, idxB)
            wait_g(idxA, rowsA, semGA)
            pltpu.async_copy(rowsA, out_ref(2 * i), semWA)
            issue_g(idxB, rowsB, semGB)
            pltpu.make_async_copy(rowsA, out_ref(2 * i), semWA).wait()

            @pl.when(2 * i + 2 < CPT_P)
            def _():
                stage(2 * i + 2, idxA)
                issue_g(idxA, rowsA, semGA)

            wait_g(idxB, rowsB, semGB)
            pltpu.async_copy(rowsB, out_ref(2 * i + 1), semWB)
            pltpu.make_async_copy(rowsB, out_ref(2 * i + 1), semWB).wait()
            return 0

        lax.fori_loop(0, CPT_P // 2, pair, 0)

    @pl.when(c == 0)
    def _():
        run(sh_tab, xsh, 0)

    @pl.when(c == 1)
    def _():
        run(co_tab, xco, EMB)


def _emb(sh_tab, co_tab, xsh2d, xco2d):
    return pl.kernel(
        _emb_body,
        out_type=jax.ShapeDtypeStruct((NPOOL, 128), F32),
        mesh=_MESH,
        compiler_params=_SC_PARAMS,
        scratch_types=[
            pltpu.VMEM((GPC, GL), jnp.int32),
            pltpu.VMEM((GPC, GL), jnp.int32),
            pltpu.VMEM((CHUNK, EMB), F32),
            pltpu.VMEM((CHUNK, EMB), F32),
            pltpu.SemaphoreType.DMA,
            pltpu.SemaphoreType.DMA,
            pltpu.SemaphoreType.DMA,
            pltpu.SemaphoreType.DMA,
        ],
    )(sh_tab, co_tab, xsh2d, xco2d)


def _agg_body(src0, dst0, src1, dst1, src2, dst2, dstp, h8,
              o0, o1, o2, og,
              sidxA, didxA, sidxB, didxB, rowsA, rowsB, zbuf, acc,
              semGA, semGB, semSA, semSB, with_counts):
    c = lax.axis_index("c")
    s = lax.axis_index("s")
    zero16 = jnp.zeros((16,), F32)
    _fill(zbuf, ZR, QW, zero16)
    e0 = jnp.where(lax.iota(jnp.int32, 16) == 0, 1.0, 0.0)
    srcs = (src0, src1, src2)
    dsts = (dst0, dst1, dst2)
    outs = (o0, o1, o2)

    def zero_acc():
        for k in range(5):
            pltpu.sync_copy(zbuf, acc.at[pl.ds(s * NPT + k * ZR, ZR), :])
        plsc.subcore_barrier()

    def drain(out, col):
        plsc.subcore_barrier()
        pltpu.sync_copy(acc.at[pl.ds(s * NPT, NPT), :],
                        out.at[pl.ds(s * NPT, NPT), pl.ds(col, QW)])
        plsc.subcore_barrier()

    def run(base_col):
        for r in range(3):
            for col in (base_col, base_col + QW):
                qq = col // QW

                def stage(ch, sidx, didx):
                    row0 = (s * CPT_E + ch) * GPC
                    pltpu.sync_copy(srcs[r].at[pl.ds(row0, GPC), :], sidx)
                    pltpu.sync_copy(dsts[r].at[pl.ds(row0, GPC), :], didx)
                    def xf(i, _):
                        for k in range(GL // 16):
                            v = sidx[i, pl.ds(k * 16, 16)]
                            sidx[i, pl.ds(k * 16, 16)] = v * 8 + qq
                        return 0
                    lax.fori_loop(0, GPC, xf, 0)

                def issue_g(sidx, rows, sem):
                    for j in range(GPC):
                        pltpu.async_copy(
                            h8.at[sidx.at[j]],
                            rows.at[pl.ds(j * GL, GL), :], sem)

                def wait_g(sidx, rows, sem):
                    for j in range(GPC):
                        pltpu.make_async_copy(
                            h8.at[sidx.at[j]],
                            rows.at[pl.ds(j * GL, GL), :], sem).wait()

                def issue_s(didx, rows, sem):
                    for j in range(GPC):
                        pltpu.async_copy(rows.at[pl.ds(j * GL, GL), :],
                                         acc.at[didx.at[j]], sem, add=True)

                def wait_s(didx, rows, sem):
                    for j in range(GPC):
                        pltpu.make_async_copy(
                            rows.at[pl.ds(j * GL, GL), :],
                            acc.at[didx.at[j]], sem).wait()

                zero_acc()
                stage(0, sidxA, didxA)
                issue_g(sidxA, rowsA, semGA)

                def pair(i, _):
                    stage(2 * i + 1, sidxB, didxB)
                    wait_g(sidxA, rowsA, semGA)
                    issue_s(didxA, rowsA, semSA)
                    issue_g(sidxB, rowsB, semGB)
                    wait_s(didxA, rowsA, semSA)

                    @pl.when(2 * i + 2 < CPT_E)
                    def _():
                        stage(2 * i + 2, sidxA, didxA)
                        issue_g(sidxA, rowsA, semGA)

                    wait_g(sidxB, rowsB, semGB)
                    issue_s(didxB, rowsB, semSB)
                    wait_s(didxB, rowsB, semSB)
                    return 0

                lax.fori_loop(0, CPT_E // 2, pair, 0)
                drain(outs[r], col)

    def cnt_run(dst2d, out):
        _fill(rowsA, GL, QW, e0)

        def stage(ch, didx):
            row0 = (s * CPT_E + ch) * GPC
            pltpu.sync_copy(dst2d.at[pl.ds(row0, GPC), :], didx)

        def issue_s(didx, sem):
            for j in range(GPC):
                pltpu.async_copy(rowsA.at[pl.ds(0, GL), :],
                                 acc.at[didx.at[j]], sem, add=True)

        def wait_s(didx, sem):
            for j in range(GPC):
                pltpu.make_async_copy(rowsA.at[pl.ds(0, GL), :],
                                      acc.at[didx.at[j]], sem).wait()

        zero_acc()
        stage(0, didxA)
        issue_s(didxA, semSA)

        def pair(i, _):
            stage(2 * i + 1, didxB)
            wait_s(didxA, semSA)
            issue_s(didxB, semSB)

            @pl.when(2 * i + 2 < CPT_E)
            def _():
                stage(2 * i + 2, didxA)
                issue_s(didxA, semSA)

            wait_s(didxB, semSB)
            return 0

        lax.fori_loop(0, CPT_E // 2, pair, 0)
        drain(out, CCOL)

    def cnt_batch():
        _fill(rowsA, GL, QW, e0)
        pltpu.sync_copy(zbuf.at[pl.ds(0, 65), :],
                        acc.at[pl.ds(s * 65, 65), :])
        plsc.subcore_barrier()

        def chunk(i, _):
            pltpu.sync_copy(dstp.at[pl.ds(s * 26 + 2 * i, 2), :],
                            didxA.at[pl.ds(0, 2), :])
            for j in range(2):
                pltpu.async_copy(rowsA.at[pl.ds(0, GL), :],
                                 acc.at[didxA.at[j]], semSA,
                                 add=True).wait()
            return 0

        lax.fori_loop(0, 13, chunk, 0)
        plsc.subcore_barrier()
        pltpu.sync_copy(acc.at[pl.ds(s * 64, 64), :],
                        og.at[pl.ds(s * 64, 64), pl.ds(CCOL, QW)])

    @pl.when(c == 0)
    def _():
        run(0)
        if with_counts:
            cnt_run(dst0, o0)
            cnt_run(dst1, o1)

    @pl.when(c == 1)
    def _():
        run(2 * QW)
        if with_counts:
            cnt_run(dst2, o2)
            cnt_batch()


def _agg(edges2d, dstp2d, h8, with_counts):
    def body(*refs):
        _agg_body(*refs, with_counts=with_counts)

    return pl.kernel(
        body,
        out_type=[jax.ShapeDtypeStruct((N, 128), F32)] * 3
        + [jax.ShapeDtypeStruct((NG, 128), F32)],
        mesh=_MESH,
        compiler_params=_SC_PARAMS,
        scratch_types=[
            pltpu.VMEM((GPC, GL), jnp.int32),
            pltpu.VMEM((GPC, GL), jnp.int32),
            pltpu.VMEM((GPC, GL), jnp.int32),
            pltpu.VMEM((GPC, GL), jnp.int32),
            pltpu.VMEM((CHUNK, QW), F32),
            pltpu.VMEM((CHUNK, QW), F32),
            pltpu.VMEM((ZR, QW), F32),
            pltpu.VMEM_SHARED((N_ACC, QW), F32),
            pltpu.SemaphoreType.DMA,
            pltpu.SemaphoreType.DMA,
            pltpu.SemaphoreType.DMA,
            pltpu.SemaphoreType.DMA,
        ],
    )(edges2d[0][0], edges2d[0][1], edges2d[1][0], edges2d[1][1],
      edges2d[2][0], edges2d[2][1], dstp2d, h8)


def _pool_body(dstp, h128, o0, o1, didx, rows, zbuf, acc, sem, sem2):
    c = lax.axis_index("c")
    s = lax.axis_index("s")
    wid = c * 16 + s
    _fill(zbuf, 65, 128, jnp.zeros((16,), F32))
    pltpu.sync_copy(zbuf, acc.at[pl.ds(s * 65, 65), :])
    plsc.subcore_barrier()

    def chunk(i, _):
        ch = wid * CPT_B + i
        pltpu.async_copy(h128.at[pl.ds(ch * GL, GL), :], rows, sem)
        pltpu.sync_copy(dstp.at[pl.ds(ch, 1), :], didx)
        pltpu.make_async_copy(h128.at[pl.ds(ch * GL, GL), :], rows,
                              sem).wait()
        pltpu.async_copy(rows, acc.at[didx.at[0]], sem2, add=True).wait()
        return 0

    lax.fori_loop(0, CPT_B, chunk, 0)
    plsc.subcore_barrier()

    @pl.when(c == 0)
    def _():
        pltpu.sync_copy(acc.at[pl.ds(s * 64, 64), :],
                        o0.at[pl.ds(s * 64, 64), :])

    @pl.when(c == 1)
    def _():
        pltpu.sync_copy(acc.at[pl.ds(s * 64, 64), :],
                        o1.at[pl.ds(s * 64, 64), :])


def _pool(dstp2d, h128):
    out = jax.ShapeDtypeStruct((NG, 128), F32)
    return pl.kernel(
        _pool_body,
        out_type=[out, out],
        mesh=_MESH,
        compiler_params=_SC_PARAMS,
        scratch_types=[
            pltpu.VMEM((1, GL), jnp.int32),
            pltpu.VMEM((GL, 128), F32),
            pltpu.VMEM((65, 128), F32),
            pltpu.VMEM_SHARED((NG_ACC, 128), F32),
            pltpu.SemaphoreType.DMA,
            pltpu.SemaphoreType.DMA,
        ],
    )(dstp2d, h128)


def _pad128(y):
    return jnp.concatenate(
        [y, jnp.zeros((y.shape[0], 128 - y.shape[1]), F32)], axis=1)


def _mlp0_body(x, w, b, out):
    out[...] = _pad128(jax.nn.relu(
        jnp.dot(x[...][:, :HID], w[...], preferred_element_type=F32)
        + b[...]))


def _mlp0(emb128, wT, b):
    grid = N // BN
    return pl.pallas_call(
        _mlp0_body,
        grid=(grid,),
        in_specs=[pl.BlockSpec((BN, 128), lambda i: (i, 0)),
                  pl.BlockSpec((2 * EMB, HID), lambda i: (0, 0)),
                  pl.BlockSpec((1, HID), lambda i: (0, 0))],
        out_specs=pl.BlockSpec((BN, 128), lambda i: (i, 0)),
        out_shape=jax.ShapeDtypeStruct((H_ROWS, 128), F32),
    )(emb128, wT, b.reshape(1, HID))


def _layer_body(a0, a1, a2, c0, c1, c2, h, w0, w1, w2, wr, b, out):
    y = jnp.dot(a0[...][:, :HID], w0[...], preferred_element_type=F32) \
        * (1.0 / jnp.maximum(c0[...][:, CCOL:CCOL + 1], 1.0))
    y += jnp.dot(a1[...][:, :HID], w1[...], preferred_element_type=F32) \
        * (1.0 / jnp.maximum(c1[...][:, CCOL:CCOL + 1], 1.0))
    y += jnp.dot(a2[...][:, :HID], w2[...], preferred_element_type=F32) \
        * (1.0 / jnp.maximum(c2[...][:, CCOL:CCOL + 1], 1.0))
    y += jnp.dot(h[...][:, :HID], wr[...], preferred_element_type=F32) \
        + b[...]
    out[...] = _pad128(jax.nn.relu(y))


def _layer(aggs, cnt_srcs, h128, wTs, wrT, b_sum):
    grid = N // BN
    feat = pl.BlockSpec((BN, 128), lambda i: (i, 0))
    wspec = pl.BlockSpec((HID, HID), lambda i: (0, 0))
    return pl.pallas_call(
        _layer_body,
        grid=(grid,),
        in_specs=[feat, feat, feat, feat, feat, feat, feat,
                  wspec, wspec, wspec, wspec,
                  pl.BlockSpec((1, HID), lambda i: (0, 0))],
        out_specs=feat,
        out_shape=jax.ShapeDtypeStruct((H_ROWS, 128), F32),
    )(*aggs, *cnt_srcs, h128, *wTs, wrT, b_sum.reshape(1, HID))


def _readout_body(p0, p1, g, w, b, out):
    rc = 1.0 / jnp.maximum(g[...][:, CCOL:CCOL + 1], 1.0)
    pooled = (p0[...] + p1[...])[:, :HID] * rc
    out[...] = jnp.dot(pooled, w[...], preferred_element_type=F32) + b[...]


def _readout(p0, p1, og, woutT, bout):
    return pl.pallas_call(
        _readout_body,
        in_specs=[pl.BlockSpec((NG, 128), lambda: (0, 0)),
                  pl.BlockSpec((NG, 128), lambda: (0, 0)),
                  pl.BlockSpec((NG, 128), lambda: (0, 0)),
                  pl.BlockSpec((HID, NCLS), lambda: (0, 0)),
                  pl.BlockSpec((1, NCLS), lambda: (0, 0))],
        out_specs=pl.BlockSpec((NG, NCLS), lambda: (0, 0)),
        out_shape=jax.ShapeDtypeStruct((NG, NCLS), F32),
    )(p0, p1, og, woutT, bout.reshape(1, NCLS))


def kernel(x, edge_index_seq, edge_index_shape, edge_index_color, batch,
           shape_emb, color_emb, W0, b0,
           rel1_Wl, rel1_bl, rel1_Wr,
           rel2_Wl, rel2_bl, rel2_Wr,
           Wout, bout):
    eis = (edge_index_seq, edge_index_shape, edge_index_color)

    pad_e = E_PAD - E
    pad_src = (jnp.arange(pad_e, dtype=jnp.int32) % 1024).astype(eis[0].dtype)
    pad_dst = (N + jnp.arange(pad_e, dtype=jnp.int32) % 16).astype(eis[0].dtype)
    edges2d = []
    for ei in eis:
        src = jnp.concatenate([ei[0], pad_src])
        dst = jnp.concatenate([ei[1], pad_dst])
        edges2d.append((src.reshape(EROWS, GL).astype(jnp.int32),
                        dst.reshape(EROWS, GL).astype(jnp.int32)))

    pad_n = NPOOL - N
    xi = x.astype(jnp.int32)
    xsh2d = jnp.concatenate([xi[:, 0], jnp.zeros((pad_n,), jnp.int32)]
                            ).reshape(PROWS, GL)
    xco2d = jnp.concatenate([xi[:, 1], jnp.zeros((pad_n,), jnp.int32)]
                            ).reshape(PROWS, GL)
    dstp2d = jnp.concatenate([batch.astype(jnp.int32),
                              jnp.full((H_ROWS - N,), NG, jnp.int32)]
                             ).reshape(BROWS, GL)

    emb128 = _emb(shape_emb, color_emb, xsh2d, xco2d)
    h128 = _mlp0(emb128, W0.T, b0)

    a0, a1, a2, og = _agg(edges2d, dstp2d, h128.reshape(H_ROWS * 8, QW),
                          with_counts=True)
    w1Ts = [rel1_Wl[i].T for i in range(3)]
    h128 = _layer((a0, a1, a2), (a0, a1, a2), h128, w1Ts,
                  (rel1_Wr[0] + rel1_Wr[1] + rel1_Wr[2]).T,
                  rel1_bl[0] + rel1_bl[1] + rel1_bl[2])

    b0_, b1_, b2_, _ = _agg(edges2d, dstp2d, h128.reshape(H_ROWS * 8, QW),
                            with_counts=False)
    w2Ts = [rel2_Wl[i].T for i in range(3)]
    h128 = _layer((b0_, b1_, b2_), (a0, a1, a2), h128, w2Ts,
                  (rel2_Wr[0] + rel2_Wr[1] + rel2_Wr[2]).T,
                  rel2_bl[0] + rel2_bl[1] + rel2_bl[2])

    p0, p1 = _pool(dstp2d, h128)
    return _readout(p0, p1, og, Wout.T, bout)
